# Initial kernel scaffold; baseline (speedup 1.0000x reference)
import jax
import jax.numpy as jnp
from jax.experimental import pallas as pl


def kernel(particles, log_weights, uniforms):
    raise NotImplementedError("write your pallas kernel here")


# speedup vs baseline: 5.4470x; 5.4470x over previous
import functools

import jax
import jax.numpy as jnp
from jax import lax
from jax.experimental import pallas as pl
from jax.experimental.pallas import tpu as pltpu
from jax.experimental.pallas import tpu_sc as plsc

_N = 262144
_D = 64
_NC = 2
_NS = 16
_L = 16


def _build(n, d):
    nw = _NC * _NS
    chunk = n // nw
    blk = _L
    nb = n // blk
    cb_w = nb // nw
    uc = 128
    rounds = chunk // uc
    csteps = max(1, nb.bit_length() - 1)

    mesh = plsc.VectorSubcoreMesh(
        core_axis_name="c", subcore_axis_name="s",
        num_cores=_NC, num_subcores=_NS)

    def wid():
        return lax.axis_index("s") * _NC + lax.axis_index("c")

    @functools.partial(
        pl.kernel,
        out_type=jax.ShapeDtypeStruct((nb,), jnp.float32),
        mesh=mesh,
        scratch_types=[pltpu.VMEM((chunk,), jnp.float32),
                       pltpu.VMEM((cb_w,), jnp.float32)],
        compiler_params=pltpu.CompilerParams(needs_layout_passes=False,
                                             use_tc_tiling_on_sc=False),
    )
    def coarse_kernel(cdf_hbm, co_hbm, cdf_v, co_v):
        w = wid()
        pltpu.sync_copy(cdf_hbm.at[pl.ds(w * chunk, chunk)], cdf_v)
        iota = lax.iota(jnp.int32, _L)
        def co(g, _):
            idx = (iota + g * _L) * blk + (blk - 1)
            co_v[pl.ds(g * _L, _L)] = plsc.load_gather(cdf_v, [idx])
            return 0
        lax.fori_loop(0, cb_w // _L, co, 0)
        pltpu.sync_copy(co_v, co_hbm.at[pl.ds(w * cb_w, cb_w)])

    @functools.partial(
        pl.kernel,
        out_type=jax.ShapeDtypeStruct((n, d), jnp.float32),
        mesh=mesh,
        scratch_types=[pltpu.VMEM((chunk,), jnp.float32),
                       pltpu.VMEM((nb,), jnp.float32),
                       pltpu.VMEM((chunk,), jnp.int32),
                       pltpu.VMEM((chunk,), jnp.int32),
                       pltpu.VMEM((uc, blk), jnp.float32),
                       pltpu.VMEM((2, uc, d), jnp.float32),
                       pltpu.SemaphoreType.DMA,
                       pltpu.SemaphoreType.DMA],
        compiler_params=pltpu.CompilerParams(needs_layout_passes=False,
                                             use_tc_tiling_on_sc=False),
    )
    def resample_kernel(part_hbm, u_hbm, cdf2_hbm, co_hbm, out_hbm,
                        u_v, co_v, bidx_v, fidx_v, fine_v, rows_v,
                        fsem, rsem):
        w = wid()
        pltpu.sync_copy(u_hbm.at[pl.ds(w * chunk, chunk)], u_v)
        pltpu.sync_copy(co_hbm, co_v)
        iota = lax.iota(jnp.int32, _L)
        def cg(g, _):
            u16 = u_v[pl.ds(g * _L, _L)]
            pos = jnp.zeros((_L,), jnp.int32)
            for sstep in [1 << (csteps - 1 - j) for j in range(csteps)]:
                val = plsc.load_gather(co_v, [pos + (sstep - 1)])
                pos = pos + jnp.where(val < u16, sstep, 0)
            bidx_v[pl.ds(g * _L, _L)] = jnp.minimum(pos, nb - 1)
            return 0
        lax.fori_loop(0, chunk // _L, cg, 0)
        def fine_round(k, _):
            pltpu.async_copy(
                cdf2_hbm.at[bidx_v.at[pl.ds(k * uc, uc)]], fine_v, fsem
            ).wait()
            def fg(g2, _):
                base = g2 * _L
                u16 = u_v[pl.ds(k * uc + base, _L)]
                b16 = bidx_v[pl.ds(k * uc + base, _L)]
                rows = iota + base
                cpos = jnp.zeros((_L,), jnp.int32)
                for sstep in (8, 4, 2, 1):
                    val = plsc.load_gather(fine_v, [rows, cpos + (sstep - 1)])
                    cpos = cpos + jnp.where(val < u16, sstep, 0)
                fidx_v[pl.ds(k * uc + base, _L)] = jnp.minimum(
                    b16 * blk + cpos, n - 1)
                return 0
            lax.fori_loop(0, uc // _L, fg, 0)
            return 0
        lax.fori_loop(0, rounds, fine_round, 0)
        def grow(kk, _):
            for b in range(2):
                k = kk * 2 + b
                pltpu.async_copy(
                    part_hbm.at[fidx_v.at[pl.ds(k * uc, uc)]],
                    rows_v.at[b], rsem).wait()
                pltpu.sync_copy(rows_v.at[b],
                                out_hbm.at[pl.ds(w * chunk + k * uc, uc)])
            return 0
        lax.fori_loop(0, rounds // 2, grow, 0)

    def run(particles, log_weights, uniforms):
        norm_lw = log_weights - jax.scipy.special.logsumexp(log_weights)
        weights = jnp.exp(norm_lw)
        cdf = jnp.cumsum(weights)
        cdf = cdf / cdf[-1]
        coarse = coarse_kernel(cdf)
        return resample_kernel(particles, uniforms,
                               cdf.reshape(nb, blk), coarse)

    return run


_run = None


def kernel(particles, log_weights, uniforms):
    global _run
    if _run is None:
        _run = _build(_N, _D)
    return _run(particles, log_weights, uniforms)
